# Initial kernel scaffold; baseline (speedup 1.0000x reference)
import functools

import jax
import jax.numpy as jnp
from jax import lax
from jax.experimental import pallas as pl
from jax.experimental.pallas import tpu as pltpu
from jax.experimental.pallas import tpu_sc as plsc

N = 100000
E = 6400000
D = 3

NP = 100352
NW = 32
B = 2048
K = B // 128
NCHUNK = E // B
GMAX = -(-NCHUNK // NW)
SEG = NP // 16


def _sc_body(pos4_hbm, ei_hbm, ew_hbm, interm, maxima,
             shpos, idxr, idxc, rrows, crows, ewbuf, outflat, accbuf, zbuf):
    cid = lax.axis_index("c")
    sid = lax.axis_index("s")
    wid = sid * 2 + cid

    pltpu.sync_copy(pos4_hbm.at[pl.ds(sid * SEG, SEG), :],
                    shpos.at[pl.ds(sid * SEG, SEG), :])
    plsc.subcore_barrier()

    accbuf[...] = jnp.zeros((16,), jnp.float32)
    zbuf[...] = jnp.zeros((16,), jnp.float32)

    iota = lax.iota(jnp.int32, 16)
    iota4 = iota * 4
    c0 = jnp.zeros((16,), jnp.int32)
    c1 = c0 + 1
    c2 = c0 + 2

    def chunk(g, carry):
        cidx = wid + g * NW

        @pl.when(cidx < NCHUNK)
        def _():
            base = cidx * B
            rb = cidx * K
            pltpu.sync_copy(ei_hbm.at[0, pl.ds(rb, K), :], idxr)
            pltpu.sync_copy(ei_hbm.at[1, pl.ds(rb, K), :], idxc)
            pltpu.sync_copy(ew_hbm.at[pl.ds(base, B)], ewbuf)
            pltpu.sync_copy(shpos.at[idxr], rrows)
            pltpu.sync_copy(shpos.at[idxc], crows)

            def inner(j, acc):
                e = iota + j * 16
                xr = plsc.load_gather(rrows, [e, c0])
                yr = plsc.load_gather(rrows, [e, c1])
                zr = plsc.load_gather(rrows, [e, c2])
                xc = plsc.load_gather(crows, [e, c0])
                yc = plsc.load_gather(crows, [e, c1])
                zc = plsc.load_gather(crows, [e, c2])
                dx = xc - xr
                dy = yc - yr
                dz = zc - zr
                w = ewbuf[pl.ds(j * 16, 16)]
                ob = iota4 + j * 64
                plsc.store_scatter(outflat, [ob], dx)
                plsc.store_scatter(outflat, [ob + 1], dy)
                plsc.store_scatter(outflat, [ob + 2], dz)
                plsc.store_scatter(outflat, [ob + 3], w)
                acc = jnp.maximum(acc, jnp.abs(dx))
                acc = jnp.maximum(acc, jnp.abs(dy))
                acc = jnp.maximum(acc, jnp.abs(dz))
                return acc

            cmax = lax.fori_loop(0, B // 16, inner, jnp.zeros((16,), jnp.float32))
            accbuf[...] = jnp.maximum(accbuf[...], cmax)
            pltpu.sync_copy(outflat, interm.at[pl.ds(base * 4, B * 4)])

        return carry

    lax.fori_loop(0, GMAX, chunk, 0)
    pltpu.sync_copy(accbuf, maxima.at[pl.ds(wid * 16, 16)])
    pltpu.sync_copy(zbuf, maxima.at[pl.ds(512 + wid * 16, 16)])


@jax.jit
def _sc_phase(pos4, ei, ew):
    f = pl.kernel(
        _sc_body,
        out_type=(
            jax.ShapeDtypeStruct((E * 4,), jnp.float32),
            jax.ShapeDtypeStruct((1024,), jnp.float32),
        ),
        mesh=plsc.VectorSubcoreMesh(core_axis_name="c", subcore_axis_name="s"),
        scratch_types=[
            pltpu.VMEM_SHARED((NP, 4), jnp.float32),
            pltpu.VMEM((K, 128), jnp.int32),
            pltpu.VMEM((K, 128), jnp.int32),
            pltpu.VMEM((B, 4), jnp.float32),
            pltpu.VMEM((B, 4), jnp.float32),
            pltpu.VMEM((B,), jnp.float32),
            pltpu.VMEM((B * 4,), jnp.float32),
            pltpu.VMEM((16,), jnp.float32),
            pltpu.VMEM((16,), jnp.float32),
        ],
    )
    return f(pos4, ei, ew)


def _tc_body(x_ref, m_ref, o_ref):
    mx = jnp.max(m_ref[...])
    s = 1.0 / (2.0 * mx)
    x = x_ref[...]
    col = lax.broadcasted_iota(jnp.int32, x.shape, 1)
    isw = (col & 3) == 3
    o_ref[...] = jnp.where(isw, x, x * s + 0.5)


@jax.jit
def _tc_phase(x, m):
    rows = x.shape[0]
    blk = 1000
    return pl.pallas_call(
        _tc_body,
        grid=(rows // blk,),
        in_specs=[
            pl.BlockSpec((blk, 512), lambda i: (i, 0)),
            pl.BlockSpec((8, 128), lambda i: (0, 0)),
        ],
        out_specs=pl.BlockSpec((blk, 512), lambda i: (i, 0)),
        out_shape=jax.ShapeDtypeStruct((rows, 512), jnp.float32),
    )(x, m)


def kernel(pos, edge_index, edge_weight):
    pos4 = jnp.pad(pos, ((0, NP - N), (0, 1)))
    ei = edge_index.astype(jnp.int32).reshape(2, E // 128, 128)
    interm, maxima = _sc_phase(pos4, ei, edge_weight)
    x = interm.reshape(E * 4 // 512, 512)
    m = maxima.reshape(8, 128)
    out = _tc_phase(x, m)
    return out.reshape(E, 4)


# speedup vs baseline: 9.2610x; 9.2610x over previous
import functools

import jax
import jax.numpy as jnp
from jax import lax
from jax.experimental import pallas as pl
from jax.experimental.pallas import tpu as pltpu
from jax.experimental.pallas import tpu_sc as plsc

N = 100000
E = 6400000
D = 3

NP = 100352
NW = 32
B = 2048
NCHUNK = E // B
GMAX = -(-NCHUNK // NW)
SEG = NP // 16


def _sc_body(px_hbm, py_hbm, pz_hbm, ei_hbm, ew_hbm, interm, maxima,
             shx, shy, shz, idxr, idxc, xr, yr, zr, xc, yc, zc,
             ewbuf, outflat, accbuf, zbuf):
    cid = lax.axis_index("c")
    sid = lax.axis_index("s")
    wid = sid * 2 + cid

    sl = pl.ds(sid * SEG, SEG)
    pltpu.sync_copy(px_hbm.at[sl], shx.at[sl])
    pltpu.sync_copy(py_hbm.at[sl], shy.at[sl])
    pltpu.sync_copy(pz_hbm.at[sl], shz.at[sl])
    plsc.subcore_barrier()

    accbuf[...] = jnp.zeros((16,), jnp.float32)
    zbuf[...] = jnp.zeros((16,), jnp.float32)

    iota = lax.iota(jnp.int32, 16)
    iota4 = iota * 4

    def chunk(g, carry):
        cidx = wid + g * NW

        @pl.when(cidx < NCHUNK)
        def _():
            base = cidx * B
            pltpu.sync_copy(ei_hbm.at[0, pl.ds(base, B)], idxr)
            pltpu.sync_copy(ei_hbm.at[1, pl.ds(base, B)], idxc)
            pltpu.sync_copy(ew_hbm.at[pl.ds(base, B)], ewbuf)
            pltpu.sync_copy(shx.at[idxr], xr)
            pltpu.sync_copy(shy.at[idxr], yr)
            pltpu.sync_copy(shz.at[idxr], zr)
            pltpu.sync_copy(shx.at[idxc], xc)
            pltpu.sync_copy(shy.at[idxc], yc)
            pltpu.sync_copy(shz.at[idxc], zc)

            def inner(j, acc):
                e = pl.ds(j * 16, 16)
                dx = xc[e] - xr[e]
                dy = yc[e] - yr[e]
                dz = zc[e] - zr[e]
                w = ewbuf[e]
                ob = iota4 + j * 64
                plsc.store_scatter(outflat, [ob], dx)
                plsc.store_scatter(outflat, [ob + 1], dy)
                plsc.store_scatter(outflat, [ob + 2], dz)
                plsc.store_scatter(outflat, [ob + 3], w)
                acc = jnp.maximum(acc, jnp.abs(dx))
                acc = jnp.maximum(acc, jnp.abs(dy))
                acc = jnp.maximum(acc, jnp.abs(dz))
                return acc

            cmax = lax.fori_loop(0, B // 16, inner, jnp.zeros((16,), jnp.float32))
            accbuf[...] = jnp.maximum(accbuf[...], cmax)
            pltpu.sync_copy(outflat, interm.at[pl.ds(base * 4, B * 4)])

        return carry

    lax.fori_loop(0, GMAX, chunk, 0)
    pltpu.sync_copy(accbuf, maxima.at[pl.ds(wid * 16, 16)])
    pltpu.sync_copy(zbuf, maxima.at[pl.ds(512 + wid * 16, 16)])


@jax.jit
def _sc_phase(px, py, pz, ei, ew):
    f = pl.kernel(
        _sc_body,
        out_type=(
            jax.ShapeDtypeStruct((E * 4,), jnp.float32),
            jax.ShapeDtypeStruct((1024,), jnp.float32),
        ),
        mesh=plsc.VectorSubcoreMesh(core_axis_name="c", subcore_axis_name="s"),
        compiler_params=pltpu.CompilerParams(needs_layout_passes=False),
        scratch_types=[
            pltpu.VMEM_SHARED((NP,), jnp.float32),
            pltpu.VMEM_SHARED((NP,), jnp.float32),
            pltpu.VMEM_SHARED((NP,), jnp.float32),
            pltpu.VMEM((B,), jnp.int32),
            pltpu.VMEM((B,), jnp.int32),
            pltpu.VMEM((B,), jnp.float32),
            pltpu.VMEM((B,), jnp.float32),
            pltpu.VMEM((B,), jnp.float32),
            pltpu.VMEM((B,), jnp.float32),
            pltpu.VMEM((B,), jnp.float32),
            pltpu.VMEM((B,), jnp.float32),
            pltpu.VMEM((B,), jnp.float32),
            pltpu.VMEM((B * 4,), jnp.float32),
            pltpu.VMEM((16,), jnp.float32),
            pltpu.VMEM((16,), jnp.float32),
        ],
    )
    return f(px, py, pz, ei, ew)


def _tc_body(x_ref, m_ref, o_ref):
    mx = jnp.max(m_ref[...])
    s = 1.0 / (2.0 * mx)
    x = x_ref[...]
    col = lax.broadcasted_iota(jnp.int32, x.shape, 1)
    isw = (col & 3) == 3
    o_ref[...] = jnp.where(isw, x, x * s + 0.5)


@jax.jit
def _tc_phase(x, m):
    rows = x.shape[0]
    blk = 1000
    return pl.pallas_call(
        _tc_body,
        grid=(rows // blk,),
        in_specs=[
            pl.BlockSpec((blk, 512), lambda i: (i, 0)),
            pl.BlockSpec((8, 128), lambda i: (0, 0)),
        ],
        out_specs=pl.BlockSpec((blk, 512), lambda i: (i, 0)),
        out_shape=jax.ShapeDtypeStruct((rows, 512), jnp.float32),
    )(x, m)


def kernel(pos, edge_index, edge_weight):
    pos_pad = jnp.pad(pos, ((0, NP - N), (0, 0)))
    px = pos_pad[:, 0]
    py = pos_pad[:, 1]
    pz = pos_pad[:, 2]
    ei = edge_index.astype(jnp.int32)
    interm, maxima = _sc_phase(px, py, pz, ei, edge_weight)
    x = interm.reshape(E * 4 // 512, 512)
    m = maxima.reshape(8, 128)
    out = _tc_phase(x, m)
    return out.reshape(E, 4)
